# baseline (speedup 1.0000x reference)
import functools

import jax
import jax.numpy as jnp
from jax import lax
from jax.experimental import pallas as pl
from jax.experimental.pallas import tpu as pltpu
from jax.experimental.pallas import tpu_sc as plsc

N = 20000
NUM_OBJ_CLS = 151
NUM_REL_CLS = 51
REL_DIM = 4096

BLOCK_N = 1000

NW = 32
ROWS_W = N // NW
G = 16
NGRP = (ROWS_W + G - 1) // G


def _mm_body(vr_ref, wt_ref, b_ref, rel_ref):
    rel = jnp.dot(vr_ref[...], wt_ref[...], preferred_element_type=jnp.float32)
    rel_ref[...] = rel + b_ref[...]


def _matmul(vr, wt, b2):
    return pl.pallas_call(
        _mm_body,
        grid=(N // BLOCK_N,),
        in_specs=[
            pl.BlockSpec((BLOCK_N, REL_DIM), lambda i: (i, 0)),
            pl.BlockSpec((REL_DIM, NUM_REL_CLS), lambda i: (0, 0)),
            pl.BlockSpec((1, NUM_REL_CLS), lambda i: (0, 0)),
        ],
        out_specs=pl.BlockSpec((BLOCK_N, NUM_REL_CLS), lambda i: (i, 0)),
        out_shape=jax.ShapeDtypeStruct((N, NUM_REL_CLS), jnp.float32),
        compiler_params=pltpu.CompilerParams(
            dimension_semantics=("arbitrary",),
        ),
    )(vr, wt, b2)


@functools.partial(
    pl.kernel,
    mesh=plsc.VectorSubcoreMesh(core_axis_name="c", subcore_axis_name="s"),
    compiler_params=pltpu.CompilerParams(needs_layout_passes=False),
    out_type=jax.ShapeDtypeStruct((NW, ROWS_W), jnp.int32),
    scratch_types=[
        pltpu.VMEM((ROWS_W * NUM_OBJ_CLS,), jnp.float32),
        pltpu.VMEM((ROWS_W,), jnp.int32),
    ],
)
def _sc_argmax(obj_hbm, out_hbm, slab, preds):
    w = lax.axis_index("s") * 2 + lax.axis_index("c")
    pltpu.sync_copy(obj_hbm.at[w], slab)
    lanes = lax.broadcasted_iota(jnp.int32, (G,), 0)

    def group_body(g, carry):
        r0 = jnp.minimum(g * G, ROWS_W - G)
        rows = r0 + lanes
        row_base = rows * NUM_OBJ_CLS

        def col_body(c, bc):
            best, bidx = bc
            cols = jnp.full((G,), c, jnp.int32)
            v = plsc.load_gather(slab, [row_base + cols])
            gt = v > best
            best = jnp.where(gt, v, best)
            bidx = jnp.where(gt, cols, bidx)
            return best, bidx

        init = (jnp.full((G,), -jnp.inf, jnp.float32),
                jnp.full((G,), 1, jnp.int32))
        best, bidx = lax.fori_loop(1, NUM_OBJ_CLS, col_body, init)
        plsc.store_scatter(preds, [rows], bidx)
        return carry

    lax.fori_loop(0, NGRP, group_body, 0)
    pltpu.sync_copy(preds, out_hbm.at[w])


def kernel(obj_logits, vr, W, b):
    wt = W.T
    b2 = b.reshape(1, NUM_REL_CLS)
    rel = _matmul(vr, wt, b2)
    obj3 = obj_logits.reshape(NW, ROWS_W * NUM_OBJ_CLS)
    preds = _sc_argmax(obj3).reshape(N)
    return (obj_logits, preds, rel)


# speedup vs baseline: 1.5718x; 1.5718x over previous
import functools

import jax
import jax.numpy as jnp
from jax import lax
from jax.experimental import pallas as pl
from jax.experimental.pallas import tpu as pltpu
from jax.experimental.pallas import tpu_sc as plsc

N = 20000
NUM_OBJ_CLS = 151
NUM_REL_CLS = 51
REL_DIM = 4096

BLOCK_N = 1000

NW = 32
W_STRIDE = 624
SLAB = 336
NSLAB = 2
LANES = 16
GROUPS = SLAB // LANES
CHUNKS = (0, 16, 32, 48, 64, 80, 96, 112, 128, 135)


def _mm_body(vr_ref, wt_ref, b_ref, rel_ref):
    rel = jnp.dot(vr_ref[...], wt_ref[...], preferred_element_type=jnp.float32)
    rel_ref[...] = rel + b_ref[...]


def _matmul(vr, wt, b2):
    return pl.pallas_call(
        _mm_body,
        grid=(N // BLOCK_N,),
        in_specs=[
            pl.BlockSpec((BLOCK_N, REL_DIM), lambda i: (i, 0)),
            pl.BlockSpec((REL_DIM, NUM_REL_CLS), lambda i: (0, 0)),
            pl.BlockSpec((1, NUM_REL_CLS), lambda i: (0, 0)),
        ],
        out_specs=pl.BlockSpec((BLOCK_N, NUM_REL_CLS), lambda i: (i, 0)),
        out_shape=jax.ShapeDtypeStruct((N, NUM_REL_CLS), jnp.float32),
        compiler_params=pltpu.CompilerParams(
            dimension_semantics=("arbitrary",),
        ),
    )(vr, wt, b2)


@functools.partial(
    pl.kernel,
    mesh=plsc.VectorSubcoreMesh(core_axis_name="c", subcore_axis_name="s"),
    compiler_params=pltpu.CompilerParams(needs_layout_passes=False),
    out_type=jax.ShapeDtypeStruct((N,), jnp.int32),
    scratch_types=[
        pltpu.VMEM((SLAB, NUM_OBJ_CLS), jnp.float32),
        pltpu.VMEM((SLAB,), jnp.int32),
    ],
)
def _sc_argmax(obj_hbm, out_hbm, slab, preds):
    w = lax.axis_index("s") * 2 + lax.axis_index("c")
    lanes = lax.broadcasted_iota(jnp.int32, (LANES,), 0)
    neg_inf = jnp.full((LANES,), -jnp.inf, jnp.float32)
    chunk_idx = [lanes + c0 for c0 in CHUNKS]

    for s in range(NSLAB):
        start = jnp.minimum(w * W_STRIDE + s * SLAB, N - SLAB)
        pltpu.sync_copy(obj_hbm.at[pl.ds(start, SLAB)], slab)

        def group_body(g, carry):
            r0 = g * LANES
            acc = jnp.full((LANES,), 0, jnp.int32)
            for j in range(LANES):
                best = jnp.full((LANES,), -jnp.inf, jnp.float32)
                bidx = jnp.full((LANES,), 1, jnp.int32)
                for ci, c0 in enumerate(CHUNKS):
                    v = slab[r0 + j, pl.ds(c0, LANES)]
                    if c0 == 0:
                        v = jnp.where(lanes == 0, neg_inf, v)
                    gt = v > best
                    best = jnp.where(gt, v, best)
                    bidx = jnp.where(gt, chunk_idx[ci], bidx)
                m = jnp.max(best, axis=0)
                cand = jnp.where(best == m, bidx, NUM_OBJ_CLS)
                acc = jnp.where(lanes == j, jnp.min(cand, axis=0), acc)
            preds[pl.ds(r0, LANES)] = acc
            return carry

        lax.fori_loop(0, GROUPS, group_body, 0)
        pltpu.sync_copy(preds, out_hbm.at[pl.ds(start, SLAB)])


def kernel(obj_logits, vr, W, b):
    wt = W.T
    b2 = b.reshape(1, NUM_REL_CLS)
    rel = _matmul(vr, wt, b2)
    preds = _sc_argmax(obj_logits)
    return (obj_logits, preds, rel)


# speedup vs baseline: 1.5723x; 1.0004x over previous
import functools

import jax
import jax.numpy as jnp
from jax import lax
from jax.experimental import pallas as pl
from jax.experimental.pallas import tpu as pltpu
from jax.experimental.pallas import tpu_sc as plsc

N = 20000
NUM_OBJ_CLS = 151
NUM_REL_CLS = 51
REL_DIM = 4096

BLOCK_N = 1000

NW = 32
W_STRIDE = 624
SLAB = 336
NSLAB = 2
LANES = 16
GROUPS = SLAB // LANES
CHUNKS = (0, 16, 32, 48, 64, 80, 96, 112, 128, 135)


def _mm_body(vr_ref, wt_ref, b_ref, rel_ref):
    rel = jnp.dot(vr_ref[...], wt_ref[...], preferred_element_type=jnp.float32)
    rel_ref[...] = rel + b_ref[...]


def _matmul(vr, wt, b2):
    return pl.pallas_call(
        _mm_body,
        grid=(N // BLOCK_N,),
        in_specs=[
            pl.BlockSpec((BLOCK_N, REL_DIM), lambda i: (i, 0)),
            pl.BlockSpec((REL_DIM, NUM_REL_CLS), lambda i: (0, 0)),
            pl.BlockSpec((1, NUM_REL_CLS), lambda i: (0, 0)),
        ],
        out_specs=pl.BlockSpec((BLOCK_N, NUM_REL_CLS), lambda i: (i, 0)),
        out_shape=jax.ShapeDtypeStruct((N, NUM_REL_CLS), jnp.float32),
        compiler_params=pltpu.CompilerParams(
            dimension_semantics=("arbitrary",),
        ),
    )(vr, wt, b2)


@functools.partial(
    pl.kernel,
    mesh=plsc.VectorSubcoreMesh(
        core_axis_name="c", subcore_axis_name="s", num_cores=2, num_subcores=16
    ),
    compiler_params=pltpu.CompilerParams(needs_layout_passes=False),
    out_type=jax.ShapeDtypeStruct((N,), jnp.int32),
    scratch_types=[
        pltpu.VMEM((SLAB, NUM_OBJ_CLS), jnp.float32),
        pltpu.VMEM((SLAB,), jnp.int32),
    ],
)
def _sc_argmax(obj_hbm, out_hbm, slab, preds):
    w = lax.axis_index("s") * 2 + lax.axis_index("c")
    lanes = lax.broadcasted_iota(jnp.int32, (LANES,), 0)
    neg_inf = jnp.full((LANES,), -jnp.inf, jnp.float32)
    chunk_idx = [lanes + c0 for c0 in CHUNKS]

    for s in range(NSLAB):
        start = jnp.minimum(w * W_STRIDE + s * SLAB, N - SLAB)
        pltpu.sync_copy(obj_hbm.at[pl.ds(start, SLAB)], slab)

        def group_body(g, carry):
            r0 = g * LANES
            acc = jnp.full((LANES,), 0, jnp.int32)
            for j in range(LANES):
                best = jnp.full((LANES,), -jnp.inf, jnp.float32)
                bidx = jnp.full((LANES,), 1, jnp.int32)
                for ci, c0 in enumerate(CHUNKS):
                    v = slab[r0 + j, pl.ds(c0, LANES)]
                    if c0 == 0:
                        v = jnp.where(lanes == 0, neg_inf, v)
                    gt = v > best
                    best = jnp.where(gt, v, best)
                    bidx = jnp.where(gt, chunk_idx[ci], bidx)
                m = jnp.max(best, axis=0)
                cand = jnp.where(best == m, bidx, NUM_OBJ_CLS)
                acc = jnp.where(lanes == j, jnp.min(cand, axis=0), acc)
            preds[pl.ds(r0, LANES)] = acc
            return carry

        lax.fori_loop(0, GROUPS, group_body, 0)
        pltpu.sync_copy(preds, out_hbm.at[pl.ds(start, SLAB)])


def kernel(obj_logits, vr, W, b):
    wt = W.T
    b2 = b.reshape(1, NUM_REL_CLS)
    preds = _sc_argmax(obj_logits)
    rel = _matmul(vr, wt, b2)
    return (obj_logits, preds, rel)


# speedup vs baseline: 1.5735x; 1.0008x over previous
import functools

import jax
import jax.numpy as jnp
from jax import lax
from jax.experimental import pallas as pl
from jax.experimental.pallas import tpu as pltpu
from jax.experimental.pallas import tpu_sc as plsc

N = 20000
NUM_OBJ_CLS = 151
NUM_REL_CLS = 51
REL_DIM = 4096

BLOCK_N = 1000

NW = 32
W_STRIDE = 624
SLAB = 336
NSLAB = 2
LANES = 16
GROUPS = SLAB // LANES
CHUNKS = (0, 16, 32, 48, 64, 80, 96, 112, 128, 135)


def _mm_body(vr_ref, wt_ref, b_ref, rel_ref):
    rel = jnp.dot(vr_ref[...], wt_ref[...], preferred_element_type=jnp.float32)
    rel_ref[...] = rel + b_ref[...]


def _matmul(vr, wt, b2):
    return pl.pallas_call(
        _mm_body,
        grid=(N // BLOCK_N,),
        in_specs=[
            pl.BlockSpec((BLOCK_N, REL_DIM), lambda i: (i, 0)),
            pl.BlockSpec((REL_DIM, NUM_REL_CLS), lambda i: (0, 0)),
            pl.BlockSpec((1, NUM_REL_CLS), lambda i: (0, 0)),
        ],
        out_specs=pl.BlockSpec((BLOCK_N, NUM_REL_CLS), lambda i: (i, 0)),
        out_shape=jax.ShapeDtypeStruct((N, NUM_REL_CLS), jnp.float32),
        compiler_params=pltpu.CompilerParams(
            dimension_semantics=("parallel",),
        ),
    )(vr, wt, b2)


@functools.partial(
    pl.kernel,
    mesh=plsc.VectorSubcoreMesh(
        core_axis_name="c", subcore_axis_name="s", num_cores=2, num_subcores=16
    ),
    compiler_params=pltpu.CompilerParams(needs_layout_passes=False),
    out_type=jax.ShapeDtypeStruct((N,), jnp.int32),
    scratch_types=[
        pltpu.VMEM((SLAB, NUM_OBJ_CLS), jnp.float32),
        pltpu.VMEM((SLAB,), jnp.int32),
    ],
)
def _sc_argmax(obj_hbm, out_hbm, slab, preds):
    w = lax.axis_index("s") * 2 + lax.axis_index("c")
    lanes = lax.broadcasted_iota(jnp.int32, (LANES,), 0)
    neg_inf = jnp.full((LANES,), -jnp.inf, jnp.float32)
    chunk_idx = [lanes + c0 for c0 in CHUNKS]

    for s in range(NSLAB):
        start = jnp.minimum(w * W_STRIDE + s * SLAB, N - SLAB)
        pltpu.sync_copy(obj_hbm.at[pl.ds(start, SLAB)], slab)

        def group_body(g, carry):
            r0 = g * LANES
            acc = jnp.full((LANES,), 0, jnp.int32)
            for j in range(LANES):
                best = jnp.full((LANES,), -jnp.inf, jnp.float32)
                bidx = jnp.full((LANES,), 1, jnp.int32)
                for ci, c0 in enumerate(CHUNKS):
                    v = slab[r0 + j, pl.ds(c0, LANES)]
                    if c0 == 0:
                        v = jnp.where(lanes == 0, neg_inf, v)
                    gt = v > best
                    best = jnp.where(gt, v, best)
                    bidx = jnp.where(gt, chunk_idx[ci], bidx)
                m = jnp.max(best, axis=0)
                cand = jnp.where(best == m, bidx, NUM_OBJ_CLS)
                acc = jnp.where(lanes == j, jnp.min(cand, axis=0), acc)
            preds[pl.ds(r0, LANES)] = acc
            return carry

        lax.fori_loop(0, GROUPS, group_body, 0)
        pltpu.sync_copy(preds, out_hbm.at[pl.ds(start, SLAB)])


def kernel(obj_logits, vr, W, b):
    wt = W.T
    b2 = b.reshape(1, NUM_REL_CLS)
    preds = _sc_argmax(obj_logits)
    rel = _matmul(vr, wt, b2)
    return (obj_logits, preds, rel)
